# baseline (speedup 1.0000x reference)
import functools

import jax
import jax.numpy as jnp
from jax import lax
from jax.experimental import pallas as pl
from jax.experimental.pallas import tpu as pltpu
from jax.experimental.pallas import tpu_sc as plsc

_CH = 128
_NBUF = 8
_K = 6


@functools.lru_cache(maxsize=None)
def _make_gather(v, d, nw, per_w, s_len, b_len):
    n_ch = per_w // _CH
    mesh = plsc.VectorSubcoreMesh(core_axis_name="c", subcore_axis_name="s")

    @functools.partial(
        pl.kernel,
        out_type=jax.ShapeDtypeStruct((s_len, b_len, 2 * d), jnp.float32),
        mesh=mesh,
        compiler_params=pltpu.CompilerParams(use_tc_tiling_on_sc=False),
        scratch_types=[
            pltpu.VMEM((n_ch, _CH), jnp.int32),
            [pltpu.VMEM((_CH, d), jnp.float32) for _ in range(_NBUF)],
            [pltpu.SemaphoreType.DMA for _ in range(_NBUF)],
            [pltpu.SemaphoreType.DMA for _ in range(_NBUF)],
        ],
    )
    def gather_kernel(idx_hbm, table_hbm, out_hbm, idx_v, bufs, gsems, wsems):
        wid = lax.axis_index("s") * 2 + lax.axis_index("c")
        base = wid * per_w

        def out_slice(g):
            fb = base + g * _CH
            return out_hbm.at[fb // b_len, pl.ds(fb % b_len, _CH), pl.ds(0, d)]

        pltpu.sync_copy(idx_hbm.at[wid], idx_v)
        for g in range(_K):
            pltpu.async_copy(table_hbm.at[idx_v.at[g]], bufs[g], gsems[g])

        @pl.loop(0, n_ch, step=_NBUF)
        def _(i):
            for p in range(_NBUF):
                g = i + p
                sk = (p + _K) % _NBUF

                @pl.when(g + _K < n_ch)
                def _refill():
                    @pl.when(g + _K >= _NBUF)
                    def _drain_prev_write():
                        pltpu.make_async_copy(
                            bufs[sk], out_slice(g + _K - _NBUF), wsems[sk]
                        ).wait()

                    pltpu.async_copy(
                        table_hbm.at[idx_v.at[g + _K]], bufs[sk], gsems[sk])

                pltpu.make_async_copy(
                    table_hbm.at[idx_v.at[g]], bufs[p], gsems[p]).wait()
                pltpu.async_copy(bufs[p], out_slice(g), wsems[p])

        for p in range(_NBUF):
            pltpu.make_async_copy(
                bufs[p], out_slice(n_ch - _NBUF + p), wsems[p]).wait()

    return gather_kernel


def kernel(indexseq, table):
    b, s = indexseq.shape
    v, d = table.shape
    nw = 32
    n_rows = s * b
    per_w = n_rows // nw
    idx3 = (jnp.transpose(indexseq.astype(jnp.int32)) * 2).reshape(
        nw, per_w // _CH, _CH)
    tbl_pad = jnp.concatenate([table, table], axis=1)
    tbl_lin = jnp.reshape(tbl_pad, (2 * v, d))
    out_pad = _make_gather(2 * v, d, nw, per_w, s, b)(idx3, tbl_lin)
    return out_pad[:, :, :d]


# speedup vs baseline: 1.1869x; 1.1869x over previous
import functools

import jax
import jax.numpy as jnp
from jax import lax
from jax.experimental import pallas as pl
from jax.experimental.pallas import tpu as pltpu
from jax.experimental.pallas import tpu_sc as plsc

_CH = 128
_NBUF = 8
_K = 6


@functools.lru_cache(maxsize=None)
def _make_gather(v, d, nw, per_w, s_len, b_len):
    n_ch = per_w // _CH
    mesh = plsc.VectorSubcoreMesh(core_axis_name="c", subcore_axis_name="s")

    @functools.partial(
        pl.kernel,
        out_type=jax.ShapeDtypeStruct((s_len, b_len, 2 * d), jnp.float32),
        mesh=mesh,
        compiler_params=pltpu.CompilerParams(use_tc_tiling_on_sc=False),
        scratch_types=[
            pltpu.VMEM((n_ch, _CH), jnp.int32),
            [pltpu.VMEM((_CH, d), jnp.float32) for _ in range(_NBUF)],
            [pltpu.SemaphoreType.DMA for _ in range(_NBUF)],
            [pltpu.SemaphoreType.DMA for _ in range(_NBUF)],
        ],
    )
    def gather_kernel(idx_hbm, table_hbm, out_hbm, idx_v, bufs, gsems, wsems):
        wid = lax.axis_index("s") * 2 + lax.axis_index("c")
        base = wid * per_w

        def out_slice(g):
            fb = base + g * _CH
            return out_hbm.at[fb // b_len, pl.ds(fb % b_len, _CH), pl.ds(0, d)]

        pltpu.sync_copy(idx_hbm.at[wid], idx_v)
        for g in range(_K):
            pltpu.async_copy(table_hbm.at[idx_v.at[g]], bufs[g], gsems[g])

        @pl.loop(0, n_ch, step=_NBUF)
        def _(i):
            for p in range(_NBUF):
                g = i + p
                sk = (p + _K) % _NBUF

                @pl.when(g + _K < n_ch)
                def _refill():
                    @pl.when(g + _K >= _NBUF)
                    def _drain_prev_write():
                        pltpu.make_async_copy(
                            bufs[sk], out_slice(g + _K - _NBUF), wsems[sk]
                        ).wait()

                    pltpu.async_copy(
                        table_hbm.at[idx_v.at[g + _K]], bufs[sk], gsems[sk])

                pltpu.make_async_copy(
                    table_hbm.at[idx_v.at[g]], bufs[p], gsems[p]).wait()
                pltpu.async_copy(bufs[p], out_slice(g), wsems[p])

        for p in range(_NBUF):
            pltpu.make_async_copy(
                bufs[p], out_slice(n_ch - _NBUF + p), wsems[p]).wait()

    return gather_kernel


def kernel(indexseq, table):
    b, s = indexseq.shape
    v, d = table.shape
    nw = 32
    n_rows = s * b
    per_w = n_rows // nw
    idx3 = (jnp.transpose(indexseq.astype(jnp.int32)) * 2).reshape(
        nw, per_w // _CH, _CH)
    tbl_pad = jnp.pad(table, ((0, 0), (0, _CH - d)))
    tbl_lin = jnp.reshape(tbl_pad, (2 * v, d))
    out_pad = _make_gather(2 * v, d, nw, per_w, s, b)(idx3, tbl_lin)
    return out_pad[:, :, :d]
